# baseline (speedup 1.0000x reference)
import functools

import jax
import jax.numpy as jnp
from jax import lax
from jax.experimental import pallas as pl
from jax.experimental.pallas import tpu as pltpu
from jax.experimental.pallas import tpu_sc as plsc

N = 6000
E = 384000
H = 256
SCW = 32

NC = 2
NS = 16
NW = NC * NS
L = 16
EPW = E // NW
CG = 80
NCHUNK = EPW // CG
DG = 272


def _gather_body(p_hbm, q_hbm, eif_hbm, g_hbm,
                 rowv, colv, pg0, qg0, pg1, qg1, sp0, sq0, sp1, sq1):
    wid = lax.axis_index("s") * NC + lax.axis_index("c")
    base0 = wid * EPW
    pltpu.sync_copy(eif_hbm.at[pl.ds(base0, EPW)], rowv)
    pltpu.sync_copy(eif_hbm.at[pl.ds(E + base0, EPW)], colv)

    def fire(k, pg, qg, sp, sq):
        off = k * CG
        pltpu.async_copy(p_hbm.at[rowv.at[pl.ds(off, CG)]], pg, sp)
        pltpu.async_copy(q_hbm.at[colv.at[pl.ds(off, CG)]], qg, sq)

    def drain(k, pg, qg, sp, sq):
        off = k * CG
        pltpu.make_async_copy(p_hbm.at[rowv.at[pl.ds(off, CG)]], pg, sp).wait()
        pltpu.make_async_copy(q_hbm.at[colv.at[pl.ds(off, CG)]], qg, sq).wait()

    def addwrite(k, pg, qg):
        @plsc.parallel_loop(0, CG, step=1, unroll=4)
        def abody(j):
            for l in range(H // L):
                sl = pl.ds(l * L, L)
                pg[j, sl] = pg[j, sl] + qg[j, sl]
            sl = pl.ds(H, L)
            pg[j, sl] = pg[j, sl] - qg[j, sl]

        pltpu.sync_copy(pg, g_hbm.at[pl.ds(base0 + k * CG, CG)])

    fire(0, pg0, qg0, sp0, sq0)

    def pair_body(g, _):
        k0 = 2 * g
        k1 = 2 * g + 1
        fire(k1, pg1, qg1, sp1, sq1)
        drain(k0, pg0, qg0, sp0, sq0)
        addwrite(k0, pg0, qg0)

        @pl.when(k0 + 2 < NCHUNK)
        def _():
            fire(k0 + 2, pg0, qg0, sp0, sq0)

        drain(k1, pg1, qg1, sp1, sq1)
        addwrite(k1, pg1, qg1)
        return 0

    lax.fori_loop(0, NCHUNK // 2, pair_body, 0)


def _gather_stage(P, Q, eif):
    mesh = plsc.VectorSubcoreMesh(core_axis_name="c", subcore_axis_name="s")
    f = functools.partial(
        pl.kernel,
        out_type=jax.ShapeDtypeStruct((E, DG), jnp.float32),
        mesh=mesh,
        scratch_types=[
            pltpu.VMEM((EPW,), jnp.int32),
            pltpu.VMEM((EPW,), jnp.int32),
            pltpu.VMEM((CG, DG), jnp.float32),
            pltpu.VMEM((CG, DG), jnp.float32),
            pltpu.VMEM((CG, DG), jnp.float32),
            pltpu.VMEM((CG, DG), jnp.float32),
            pltpu.SemaphoreType.DMA,
            pltpu.SemaphoreType.DMA,
            pltpu.SemaphoreType.DMA,
            pltpu.SemaphoreType.DMA,
        ],
        compiler_params=pltpu.CompilerParams(
            needs_layout_passes=False, use_tc_tiling_on_sc=False),
    )(_gather_body)
    return f(P, Q, eif)


def _pq_body(nf_ref, wa_ref, wb_ref, eb1_ref, coords_ref, p_ref, q_ref):
    nf = nf_ref[...]
    cpad = jnp.concatenate(
        [coords_ref[...], jnp.zeros((N, DG - H - 3), jnp.float32)], axis=1)
    p = jnp.dot(nf, wa_ref[...], preferred_element_type=jnp.float32) + eb1_ref[...]
    q = jnp.dot(nf, wb_ref[...], preferred_element_type=jnp.float32)
    p_ref[...] = jnp.concatenate([p, cpad], axis=1)
    q_ref[...] = jnp.concatenate([q, cpad], axis=1)


def _compute_pq(nf, eW1a, eW1b, eb1, coords):
    return pl.pallas_call(
        _pq_body,
        out_shape=[
            jax.ShapeDtypeStruct((N, DG), jnp.float32),
            jax.ShapeDtypeStruct((N, DG), jnp.float32),
        ],
    )(nf, eW1a, eW1b, eb1, coords)


CS = 120
NCHUNK3 = EPW // CS
NPAD = 6016
RPT = NPAD // NS


def _scatter_body(scat_hbm, row_hbm, part_hbm, rowv, sv, zbuf, acc):
    cid = lax.axis_index("c")
    sid = lax.axis_index("s")
    wid = sid * NC + cid
    base0 = wid * EPW

    def zbody(j, _):
        for l in range(SCW // L):
            zbuf[j, pl.ds(l * L, L)] = jnp.zeros((L,), jnp.float32)
        return 0

    lax.fori_loop(0, RPT, zbody, 0)
    pltpu.sync_copy(zbuf, acc.at[pl.ds(sid * RPT, RPT)])
    plsc.subcore_barrier()

    def chunk_body(i, _):
        base = base0 + i * CS
        pltpu.sync_copy(row_hbm.at[pl.ds(base, CS)], rowv)
        pltpu.sync_copy(scat_hbm.at[pl.ds(base, CS)], sv)
        pltpu.sync_copy(sv, acc.at[rowv], add=True)
        return 0

    lax.fori_loop(0, NCHUNK3, chunk_body, 0)
    plsc.subcore_barrier()
    pltpu.sync_copy(acc.at[pl.ds(sid * RPT, RPT)], zbuf)
    pltpu.sync_copy(zbuf, part_hbm.at[cid, pl.ds(sid * RPT, RPT)])


def _scatter_stage(scat, row):
    mesh = plsc.VectorSubcoreMesh(core_axis_name="c", subcore_axis_name="s")
    f = functools.partial(
        pl.kernel,
        out_type=jax.ShapeDtypeStruct((NC, NPAD, SCW), jnp.float32),
        mesh=mesh,
        scratch_types=[
            pltpu.VMEM((CS,), jnp.int32),
            pltpu.VMEM((CS, SCW), jnp.float32),
            pltpu.VMEM((RPT, SCW), jnp.float32),
            pltpu.VMEM_SHARED((NPAD, SCW), jnp.float32),
        ],
        compiler_params=pltpu.CompilerParams(
            needs_layout_passes=False, use_tc_tiling_on_sc=False),
    )(_scatter_body)
    return f(scat, row)


BE = 3840


def _edge_body(g_ref, attr_ref, ew1c_ref, ew1d_ref, ew2_ref, eb2_ref,
               cw1_ref, cb1_ref, cw2_ref, few_ref, feb_ref, dew_ref, deb_ref,
               scat_ref, eemb_ref, erec_ref):
    attr = attr_ref[...]
    cd = g_ref[:, H:H + 3]
    radial = jnp.sum(cd * cd, axis=1, keepdims=True)
    pre = (g_ref[:, 0:H]
           + jnp.dot(attr, ew1c_ref[...], preferred_element_type=jnp.float32)
           + radial * ew1d_ref[...])
    hidden = jnp.maximum(pre, 0.0)
    edge_out = jnp.dot(hidden, ew2_ref[...], preferred_element_type=jnp.float32) + eb2_ref[...]
    h2 = jnp.maximum(jnp.dot(edge_out, cw1_ref[...], preferred_element_type=jnp.float32)
                     + cb1_ref[...], 0.0)
    coef = jnp.dot(h2, cw2_ref[...], preferred_element_type=jnp.float32)
    norm = jnp.sqrt(radial) + 1.0
    trans = (cd / norm) * coef
    eemb = jnp.dot(edge_out, few_ref[...], preferred_element_type=jnp.float32) + feb_ref[...]
    eemb_ref[...] = eemb
    erec_ref[...] = jnp.dot(eemb, dew_ref[...], preferred_element_type=jnp.float32) + deb_ref[...]
    scat_ref[...] = jnp.concatenate(
        [edge_out, trans, jnp.zeros((BE, SCW - 19), jnp.float32)], axis=1)


def _edge_stage(G, edge_attr, eW1c, eW1d, eW2, eb2, cW1, cb1, cW2,
                feW, feb, deW, deb):
    nblk = E // BE
    full = lambda s: pl.BlockSpec(s, lambda i: (0, 0))
    return pl.pallas_call(
        _edge_body,
        grid=(nblk,),
        in_specs=[
            pl.BlockSpec((BE, DG), lambda i: (i, 0)),
            pl.BlockSpec((BE, 16), lambda i: (i, 0)),
            full((16, H)), full((1, H)), full((H, 16)), full((1, 16)),
            full((16, 2)), full((1, 2)), full((2, 1)),
            full((16, 2)), full((1, 2)), full((2, 16)), full((1, 16)),
        ],
        out_specs=[
            pl.BlockSpec((BE, SCW), lambda i: (i, 0)),
            pl.BlockSpec((BE, 2), lambda i: (i, 0)),
            pl.BlockSpec((BE, 16), lambda i: (i, 0)),
        ],
        out_shape=[
            jax.ShapeDtypeStruct((E, SCW), jnp.float32),
            jax.ShapeDtypeStruct((E, 2), jnp.float32),
            jax.ShapeDtypeStruct((E, 16), jnp.float32),
        ],
    )(G, edge_attr, eW1c, eW1d, eW2, eb2, cW1, cb1, cW2, feW, feb, deW, deb)


def _node_body(nf_ref, part_ref, coords_ref, nw1a_ref, nw1b_ref, nw1c_ref,
               nb1_ref, nw2_ref, nb2_ref, fnw_ref, fnb_ref, dnw_ref, dnb_ref,
               nemb_ref, nrec_ref, cout_ref):
    agg = part_ref[0] + part_ref[1]
    agg_e = agg[:, 0:16]
    agg_c = agg[:, 16:19]
    coord_out = coords_ref[...] + agg_c
    cout_ref[...] = coord_out
    pre = (jnp.dot(nf_ref[...], nw1a_ref[...], preferred_element_type=jnp.float32)
           + jnp.dot(agg_e, nw1b_ref[...], preferred_element_type=jnp.float32)
           + jnp.dot(coord_out, nw1c_ref[...], preferred_element_type=jnp.float32)
           + nb1_ref[...])
    h = jnp.maximum(pre, 0.0)
    node_out = jnp.dot(h, nw2_ref[...], preferred_element_type=jnp.float32) + nb2_ref[...]
    nemb = jnp.dot(node_out, fnw_ref[...], preferred_element_type=jnp.float32) + fnb_ref[...]
    nemb_ref[...] = nemb
    nrec_ref[...] = jnp.dot(nemb, dnw_ref[...], preferred_element_type=jnp.float32) + dnb_ref[...]


def _node_stage(nf, partials, coords, nW1a, nW1b, nW1c, nb1, nW2, nb2,
                fnW, fnb, dnW, dnb):
    return pl.pallas_call(
        _node_body,
        out_shape=[
            jax.ShapeDtypeStruct((N, 2), jnp.float32),
            jax.ShapeDtypeStruct((N, 256), jnp.float32),
            jax.ShapeDtypeStruct((N, 3), jnp.float32),
        ],
    )(nf, partials, coords, nW1a, nW1b, nW1c, nb1, nW2, nb2, fnW, fnb, dnW, dnb)


BA = 768


def _adj_body(emb_ref, embt_ref, adj_ref):
    i = pl.program_id(0)
    j = pl.program_id(1)
    r = emb_ref[...]
    c = embt_ref[...]
    d0 = r[:, 0:1] - c[0:1, :]
    d1 = r[:, 1:2] - c[1:2, :]
    s = d0 * d0 + d1 * d1
    adj = jax.nn.sigmoid(3.0 * s - 1.0)
    row_ids = i * BA + lax.broadcasted_iota(jnp.int32, (BA, BA), 0)
    col_ids = j * BA + lax.broadcasted_iota(jnp.int32, (BA, BA), 1)
    adj_ref[...] = jnp.where(row_ids == col_ids, 0.0, adj)


def _adj_stage(node_emb, node_emb_t):
    nblk = pl.cdiv(N, BA)
    return pl.pallas_call(
        _adj_body,
        grid=(nblk, nblk),
        in_specs=[
            pl.BlockSpec((BA, 2), lambda i, j: (i, 0)),
            pl.BlockSpec((2, BA), lambda i, j: (0, j)),
        ],
        out_specs=pl.BlockSpec((BA, BA), lambda i, j: (i, j)),
        out_shape=jax.ShapeDtypeStruct((N, N), jnp.float32),
    )(node_emb, node_emb_t)


def kernel(node_feats, edge_index, edge_attr, coords, nW1, nb1, nW2, nb2,
           eW1, eb1, eW2, eb2, cW1, cb1, cW2, fnW, fnb, feW, feb, dnW, dnb,
           deW, deb):
    row = edge_index[0]
    eW1a = eW1[0:256]
    eW1b = eW1[256:512]
    eW1c = eW1[512:528]
    eW1d = eW1[528:529]

    P, Q = _compute_pq(node_feats, eW1a, eW1b, eb1.reshape(1, H), coords)

    G = _gather_stage(P, Q, edge_index.reshape(-1))

    scat, edge_emb, recon_edge = _edge_stage(
        G, edge_attr, eW1c, eW1d, eW2, eb2.reshape(1, 16),
        cW1, cb1.reshape(1, 2), cW2, feW, feb.reshape(1, 2), deW,
        deb.reshape(1, 16))

    partials = _scatter_stage(scat, row)[:, :N, :]

    node_emb, recon_node, coord_out = _node_stage(
        node_feats, partials, coords, nW1[0:256], nW1[256:272], nW1[272:275],
        nb1.reshape(1, H), nW2, nb2.reshape(1, 256), fnW, fnb.reshape(1, 2),
        dnW, dnb.reshape(1, 256))

    adj_pred = _adj_stage(node_emb, node_emb.T)

    return (node_emb, edge_emb, recon_node, recon_edge, adj_pred, coord_out)


# speedup vs baseline: 1.3047x; 1.3047x over previous
import functools

import jax
import jax.numpy as jnp
from jax import lax
from jax.experimental import pallas as pl
from jax.experimental.pallas import tpu as pltpu
from jax.experimental.pallas import tpu_sc as plsc

N = 6000
E = 384000
H = 256
SCW = 32

NC = 2
NS = 16
NW = NC * NS
L = 16
EPW = E // NW
CG = 48
NCHUNK = EPW // CG
DG = 384


def _gather_body(p_hbm, q_hbm, eif_hbm, g_hbm,
                 rowv, colv, pg0, qg0, pg1, qg1, sp0, sq0, sp1, sq1):
    wid = lax.axis_index("s") * NC + lax.axis_index("c")
    base0 = wid * EPW
    pltpu.sync_copy(eif_hbm.at[pl.ds(base0, EPW)], rowv)
    pltpu.sync_copy(eif_hbm.at[pl.ds(E + base0, EPW)], colv)

    def fire(k, pg, qg, sp, sq):
        off = k * CG
        pltpu.async_copy(p_hbm.at[rowv.at[pl.ds(off, CG)]], pg, sp)
        pltpu.async_copy(q_hbm.at[colv.at[pl.ds(off, CG)]], qg, sq)

    def drain(k, pg, qg, sp, sq):
        off = k * CG
        pltpu.make_async_copy(p_hbm.at[rowv.at[pl.ds(off, CG)]], pg, sp).wait()
        pltpu.make_async_copy(q_hbm.at[colv.at[pl.ds(off, CG)]], qg, sq).wait()

    def addwrite(k, pg, qg):
        @plsc.parallel_loop(0, CG, step=1, unroll=4)
        def abody(j):
            for l in range(H // L):
                sl = pl.ds(l * L, L)
                pg[j, sl] = pg[j, sl] + qg[j, sl]
            sl = pl.ds(H, L)
            pg[j, sl] = pg[j, sl] - qg[j, sl]

        pltpu.sync_copy(pg, g_hbm.at[pl.ds(base0 + k * CG, CG)])

    fire(0, pg0, qg0, sp0, sq0)

    def pair_body(g, _):
        k0 = 2 * g
        k1 = 2 * g + 1
        fire(k1, pg1, qg1, sp1, sq1)
        drain(k0, pg0, qg0, sp0, sq0)
        addwrite(k0, pg0, qg0)

        @pl.when(k0 + 2 < NCHUNK)
        def _():
            fire(k0 + 2, pg0, qg0, sp0, sq0)

        drain(k1, pg1, qg1, sp1, sq1)
        addwrite(k1, pg1, qg1)
        return 0

    lax.fori_loop(0, NCHUNK // 2, pair_body, 0)


def _gather_stage(P, Q, eif):
    mesh = plsc.VectorSubcoreMesh(core_axis_name="c", subcore_axis_name="s")
    f = functools.partial(
        pl.kernel,
        out_type=jax.ShapeDtypeStruct((E, DG), jnp.float32),
        mesh=mesh,
        scratch_types=[
            pltpu.VMEM((EPW,), jnp.int32),
            pltpu.VMEM((EPW,), jnp.int32),
            pltpu.VMEM((CG, DG), jnp.float32),
            pltpu.VMEM((CG, DG), jnp.float32),
            pltpu.VMEM((CG, DG), jnp.float32),
            pltpu.VMEM((CG, DG), jnp.float32),
            pltpu.SemaphoreType.DMA,
            pltpu.SemaphoreType.DMA,
            pltpu.SemaphoreType.DMA,
            pltpu.SemaphoreType.DMA,
        ],
        compiler_params=pltpu.CompilerParams(needs_layout_passes=False),
    )(_gather_body)
    return f(P, Q, eif)


def _pq_body(nf_ref, wa_ref, wb_ref, eb1_ref, coords_ref, p_ref, q_ref):
    nf = nf_ref[...]
    cpad = jnp.concatenate(
        [coords_ref[...], jnp.zeros((N, DG - H - 3), jnp.float32)], axis=1)
    p = jnp.dot(nf, wa_ref[...], preferred_element_type=jnp.float32) + eb1_ref[...]
    q = jnp.dot(nf, wb_ref[...], preferred_element_type=jnp.float32)
    p_ref[...] = jnp.concatenate([p, cpad], axis=1)
    q_ref[...] = jnp.concatenate([q, cpad], axis=1)


def _compute_pq(nf, eW1a, eW1b, eb1, coords):
    return pl.pallas_call(
        _pq_body,
        out_shape=[
            jax.ShapeDtypeStruct((N, DG), jnp.float32),
            jax.ShapeDtypeStruct((N, DG), jnp.float32),
        ],
    )(nf, eW1a, eW1b, eb1, coords)


CS = 120
NCHUNK3 = EPW // CS
NPAD = 6016
RPT = NPAD // NS


def _scatter_body(scat_hbm, row_hbm, part_hbm, rowv, sv, zbuf, acc):
    cid = lax.axis_index("c")
    sid = lax.axis_index("s")
    wid = sid * NC + cid
    base0 = wid * EPW

    def zbody(j, _):
        for l in range(SCW // L):
            zbuf[j, pl.ds(l * L, L)] = jnp.zeros((L,), jnp.float32)
        return 0

    lax.fori_loop(0, RPT, zbody, 0)
    pltpu.sync_copy(zbuf, acc.at[pl.ds(sid * RPT, RPT)])
    plsc.subcore_barrier()

    def chunk_body(i, _):
        base = base0 + i * CS
        pltpu.sync_copy(row_hbm.at[pl.ds(base, CS)], rowv)
        pltpu.sync_copy(scat_hbm.at[pl.ds(base, CS)], sv)
        pltpu.sync_copy(sv, acc.at[rowv], add=True)
        return 0

    lax.fori_loop(0, NCHUNK3, chunk_body, 0)
    plsc.subcore_barrier()
    pltpu.sync_copy(acc.at[pl.ds(sid * RPT, RPT)], zbuf)
    pltpu.sync_copy(zbuf, part_hbm.at[cid, pl.ds(sid * RPT, RPT)])


def _scatter_stage(scat, row):
    mesh = plsc.VectorSubcoreMesh(core_axis_name="c", subcore_axis_name="s")
    f = functools.partial(
        pl.kernel,
        out_type=jax.ShapeDtypeStruct((NC, NPAD, SCW), jnp.float32),
        mesh=mesh,
        scratch_types=[
            pltpu.VMEM((CS,), jnp.int32),
            pltpu.VMEM((CS, SCW), jnp.float32),
            pltpu.VMEM((RPT, SCW), jnp.float32),
            pltpu.VMEM_SHARED((NPAD, SCW), jnp.float32),
        ],
        compiler_params=pltpu.CompilerParams(
            needs_layout_passes=False, use_tc_tiling_on_sc=False),
    )(_scatter_body)
    return f(scat, row)


BE = 3840


def _edge_body(g_ref, attr_ref, ew1c_ref, ew1d_ref, ew2_ref, eb2_ref,
               cw1_ref, cb1_ref, cw2_ref, few_ref, feb_ref, dew_ref, deb_ref,
               scat_ref, eemb_ref, erec_ref):
    attr = attr_ref[...]
    cd = g_ref[:, H:H + 3]
    radial = jnp.sum(cd * cd, axis=1, keepdims=True)
    pre = (g_ref[:, 0:H]
           + jnp.dot(attr, ew1c_ref[...], preferred_element_type=jnp.float32)
           + radial * ew1d_ref[...])
    hidden = jnp.maximum(pre, 0.0)
    edge_out = jnp.dot(hidden, ew2_ref[...], preferred_element_type=jnp.float32) + eb2_ref[...]
    h2 = jnp.maximum(jnp.dot(edge_out, cw1_ref[...], preferred_element_type=jnp.float32)
                     + cb1_ref[...], 0.0)
    coef = jnp.dot(h2, cw2_ref[...], preferred_element_type=jnp.float32)
    norm = jnp.sqrt(radial) + 1.0
    trans = (cd / norm) * coef
    eemb = jnp.dot(edge_out, few_ref[...], preferred_element_type=jnp.float32) + feb_ref[...]
    eemb_ref[...] = eemb
    erec_ref[...] = jnp.dot(eemb, dew_ref[...], preferred_element_type=jnp.float32) + deb_ref[...]
    scat_ref[...] = jnp.concatenate(
        [edge_out, trans, jnp.zeros((BE, SCW - 19), jnp.float32)], axis=1)


def _edge_stage(G, edge_attr, eW1c, eW1d, eW2, eb2, cW1, cb1, cW2,
                feW, feb, deW, deb):
    nblk = E // BE
    full = lambda s: pl.BlockSpec(s, lambda i: (0, 0))
    return pl.pallas_call(
        _edge_body,
        grid=(nblk,),
        in_specs=[
            pl.BlockSpec((BE, DG), lambda i: (i, 0)),
            pl.BlockSpec((BE, 16), lambda i: (i, 0)),
            full((16, H)), full((1, H)), full((H, 16)), full((1, 16)),
            full((16, 2)), full((1, 2)), full((2, 1)),
            full((16, 2)), full((1, 2)), full((2, 16)), full((1, 16)),
        ],
        out_specs=[
            pl.BlockSpec((BE, SCW), lambda i: (i, 0)),
            pl.BlockSpec((BE, 2), lambda i: (i, 0)),
            pl.BlockSpec((BE, 16), lambda i: (i, 0)),
        ],
        out_shape=[
            jax.ShapeDtypeStruct((E, SCW), jnp.float32),
            jax.ShapeDtypeStruct((E, 2), jnp.float32),
            jax.ShapeDtypeStruct((E, 16), jnp.float32),
        ],
    )(G, edge_attr, eW1c, eW1d, eW2, eb2, cW1, cb1, cW2, feW, feb, deW, deb)


def _node_body(nf_ref, part_ref, coords_ref, nw1a_ref, nw1b_ref, nw1c_ref,
               nb1_ref, nw2_ref, nb2_ref, fnw_ref, fnb_ref, dnw_ref, dnb_ref,
               nemb_ref, nrec_ref, cout_ref):
    agg = part_ref[0] + part_ref[1]
    agg_e = agg[:, 0:16]
    agg_c = agg[:, 16:19]
    coord_out = coords_ref[...] + agg_c
    cout_ref[...] = coord_out
    pre = (jnp.dot(nf_ref[...], nw1a_ref[...], preferred_element_type=jnp.float32)
           + jnp.dot(agg_e, nw1b_ref[...], preferred_element_type=jnp.float32)
           + jnp.dot(coord_out, nw1c_ref[...], preferred_element_type=jnp.float32)
           + nb1_ref[...])
    h = jnp.maximum(pre, 0.0)
    node_out = jnp.dot(h, nw2_ref[...], preferred_element_type=jnp.float32) + nb2_ref[...]
    nemb = jnp.dot(node_out, fnw_ref[...], preferred_element_type=jnp.float32) + fnb_ref[...]
    nemb_ref[...] = nemb
    nrec_ref[...] = jnp.dot(nemb, dnw_ref[...], preferred_element_type=jnp.float32) + dnb_ref[...]


def _node_stage(nf, partials, coords, nW1a, nW1b, nW1c, nb1, nW2, nb2,
                fnW, fnb, dnW, dnb):
    return pl.pallas_call(
        _node_body,
        out_shape=[
            jax.ShapeDtypeStruct((N, 2), jnp.float32),
            jax.ShapeDtypeStruct((N, 256), jnp.float32),
            jax.ShapeDtypeStruct((N, 3), jnp.float32),
        ],
    )(nf, partials, coords, nW1a, nW1b, nW1c, nb1, nW2, nb2, fnW, fnb, dnW, dnb)


BA = 768


def _adj_body(emb_ref, embt_ref, adj_ref):
    i = pl.program_id(0)
    j = pl.program_id(1)
    r = emb_ref[...]
    c = embt_ref[...]
    d0 = r[:, 0:1] - c[0:1, :]
    d1 = r[:, 1:2] - c[1:2, :]
    s = d0 * d0 + d1 * d1
    adj = jax.nn.sigmoid(3.0 * s - 1.0)
    row_ids = i * BA + lax.broadcasted_iota(jnp.int32, (BA, BA), 0)
    col_ids = j * BA + lax.broadcasted_iota(jnp.int32, (BA, BA), 1)
    adj_ref[...] = jnp.where(row_ids == col_ids, 0.0, adj)


def _adj_stage(node_emb, node_emb_t):
    nblk = pl.cdiv(N, BA)
    return pl.pallas_call(
        _adj_body,
        grid=(nblk, nblk),
        in_specs=[
            pl.BlockSpec((BA, 2), lambda i, j: (i, 0)),
            pl.BlockSpec((2, BA), lambda i, j: (0, j)),
        ],
        out_specs=pl.BlockSpec((BA, BA), lambda i, j: (i, j)),
        out_shape=jax.ShapeDtypeStruct((N, N), jnp.float32),
    )(node_emb, node_emb_t)


def kernel(node_feats, edge_index, edge_attr, coords, nW1, nb1, nW2, nb2,
           eW1, eb1, eW2, eb2, cW1, cb1, cW2, fnW, fnb, feW, feb, dnW, dnb,
           deW, deb):
    row = edge_index[0]
    eW1a = eW1[0:256]
    eW1b = eW1[256:512]
    eW1c = eW1[512:528]
    eW1d = eW1[528:529]

    P, Q = _compute_pq(node_feats, eW1a, eW1b, eb1.reshape(1, H), coords)

    G = _gather_stage(P, Q, edge_index.reshape(-1))

    scat, edge_emb, recon_edge = _edge_stage(
        G, edge_attr, eW1c, eW1d, eW2, eb2.reshape(1, 16),
        cW1, cb1.reshape(1, 2), cW2, feW, feb.reshape(1, 2), deW,
        deb.reshape(1, 16))

    partials = _scatter_stage(scat, row)[:, :N, :]

    node_emb, recon_node, coord_out = _node_stage(
        node_feats, partials, coords, nW1[0:256], nW1[256:272], nW1[272:275],
        nb1.reshape(1, H), nW2, nb2.reshape(1, 256), fnW, fnb.reshape(1, 2),
        dnW, dnb.reshape(1, 256))

    adj_pred = _adj_stage(node_emb, node_emb.T)

    return (node_emb, edge_emb, recon_node, recon_edge, adj_pred, coord_out)
